# baseline (speedup 1.0000x reference)
import jax
import jax.numpy as jnp
from jax import lax
from jax.experimental import pallas as pl
from jax.experimental.pallas import tpu as pltpu
from jax.experimental.pallas import tpu_sc as plsc

_STRIDES = [8, 16, 32, 64, 128]
_THRESHOLD = 0.99
_HWS = [64, 32, 16, 8, 4]
_NLOC = [4096, 1024, 256, 64, 16]
_OFFS = [0, 4096, 5120, 5376, 5440]
_TOT = 5456
_C = 80
_OUTC = 87
_N = 16

_B = [256, 256, 128, 64, 16]
_LBPI = [4, 2, 1, 0, 0]
_UPW = [8, 2, 1]


def _group(cls_v, bb_v, ct_v, out_v, o, loc0, hw_mask, hw_shift, stride_sh):
    lanes = lax.broadcasted_iota(jnp.int32, (16,), 0)
    row = lanes + o
    vals = [cls_v[c, pl.ds(o, 16)] for c in range(_C)]
    m = vals[0]
    for c in range(1, _C):
        m = jnp.maximum(m, vals[c])
    mask = jnp.where(m > _THRESHOLD, 1.0, 0.0)
    loc = loc0 + row
    xs = ((loc & hw_mask) << stride_sh).astype(jnp.float32) * mask
    ys = ((loc >> hw_shift) << stride_sh).astype(jnp.float32) * mask
    plsc.store_scatter(out_v, [row, jnp.full((16,), 0, jnp.int32)], xs)
    plsc.store_scatter(out_v, [row, jnp.full((16,), 1, jnp.int32)], ys)
    for c in range(_C):
        plsc.store_scatter(out_v, [row, jnp.full((16,), 2 + c, jnp.int32)],
                           vals[c] * mask)
    for c in range(4):
        plsc.store_scatter(out_v, [row, jnp.full((16,), 82 + c, jnp.int32)],
                           bb_v[c, pl.ds(o, 16)] * mask)
    plsc.store_scatter(out_v, [row, jnp.full((16,), 86, jnp.int32)],
                       ct_v[0, pl.ds(o, 16)] * mask)


def _unit(l, u, cls_h, bb_h, ct_h, out_h, cls_v, bb_v, ct_v, out_v):
    b = _B[l]
    n = lax.shift_right_logical(u, _LBPI[l])
    blk = lax.bitwise_and(u, (1 << _LBPI[l]) - 1)
    loc0 = blk * b
    stride_sh = _STRIDES[l].bit_length() - 1
    hw_sh = _HWS[l].bit_length() - 1
    if b == _NLOC[l]:
        pltpu.sync_copy(cls_h.at[n], cls_v)
        pltpu.sync_copy(bb_h.at[n], bb_v)
        pltpu.sync_copy(ct_h.at[n], ct_v)
    else:
        pltpu.sync_copy(cls_h.at[n, :, pl.ds(loc0, b)], cls_v.at[:, pl.ds(0, b)])
        pltpu.sync_copy(bb_h.at[n, :, pl.ds(loc0, b)], bb_v.at[:, pl.ds(0, b)])
        pltpu.sync_copy(ct_h.at[n, :, pl.ds(loc0, b)], ct_v.at[:, pl.ds(0, b)])

    def group_body(g, carry):
        _group(cls_v, bb_v, ct_v, out_v, g * 16, loc0,
               _HWS[l] - 1, hw_sh, stride_sh)
        return carry

    lax.fori_loop(0, b // 16, group_body, 0, unroll=False)
    row0 = _OFFS[l] + loc0
    pltpu.sync_copy(out_v.at[pl.ds(0, b), :], out_h.at[n, pl.ds(row0, b), :])


def _sc_body(c0, c1, c2, c3, c4, b0, b1, b2, b3, b4, t0, t1, t2, t3, t4,
             out_h, cls_v, bb_v, ct_v, out_v,
             cls_v3, bb_v3, ct_v3, cls_v4, bb_v4, ct_v4):
    cls_hs = [c0, c1, c2, c3, c4]
    bb_hs = [b0, b1, b2, b3, b4]
    ct_hs = [t0, t1, t2, t3, t4]
    wid = lax.axis_index("s") * 2 + lax.axis_index("c")
    for l in range(3):
        def unit_body(k, carry, l=l):
            _unit(l, wid * _UPW[l] + k, cls_hs[l], bb_hs[l], ct_hs[l], out_h,
                  cls_v, bb_v, ct_v, out_v)
            return carry
        lax.fori_loop(0, _UPW[l], unit_body, 0, unroll=False)
    @pl.when(wid < 16)
    def _():
        _unit(3, wid, cls_hs[3], bb_hs[3], ct_hs[3], out_h,
              cls_v3, bb_v3, ct_v3, out_v)

    @pl.when(wid >= 16)
    def _():
        _unit(4, wid - 16, cls_hs[4], bb_hs[4], ct_hs[4], out_h,
              cls_v4, bb_v4, ct_v4, out_v)


def kernel(cls_scores_0, cls_scores_1, cls_scores_2, cls_scores_3, cls_scores_4,
           bbox_preds_0, bbox_preds_1, bbox_preds_2, bbox_preds_3, bbox_preds_4,
           centernesses_0, centernesses_1, centernesses_2, centernesses_3,
           centernesses_4):
    cls_l = [cls_scores_0, cls_scores_1, cls_scores_2, cls_scores_3, cls_scores_4]
    bbox_l = [bbox_preds_0, bbox_preds_1, bbox_preds_2, bbox_preds_3, bbox_preds_4]
    ctr_l = [centernesses_0, centernesses_1, centernesses_2, centernesses_3,
             centernesses_4]
    args = []
    for lst, ch in ((cls_l, _C), (bbox_l, 4), (ctr_l, 1)):
        for l in range(5):
            args.append(lst[l].reshape(_N, ch, _NLOC[l]))
    mesh = plsc.VectorSubcoreMesh(core_axis_name="c", subcore_axis_name="s")
    f = pl.kernel(
        _sc_body,
        out_type=jax.ShapeDtypeStruct((_N, _TOT, _OUTC), jnp.float32),
        mesh=mesh,
        scratch_types=[
            pltpu.VMEM((_C, 256), jnp.float32),
            pltpu.VMEM((4, 256), jnp.float32),
            pltpu.VMEM((1, 256), jnp.float32),
            pltpu.VMEM((256, _OUTC), jnp.float32),
            pltpu.VMEM((_C, 64), jnp.float32),
            pltpu.VMEM((4, 64), jnp.float32),
            pltpu.VMEM((1, 64), jnp.float32),
            pltpu.VMEM((_C, 16), jnp.float32),
            pltpu.VMEM((4, 16), jnp.float32),
            pltpu.VMEM((1, 16), jnp.float32),
        ],
        compiler_params=pltpu.CompilerParams(use_tc_tiling_on_sc=True,
                                             needs_layout_passes=False),
    )
    return f(*args)


# speedup vs baseline: 2.0341x; 2.0341x over previous
import jax
import jax.numpy as jnp
from jax.experimental import pallas as pl
from jax.experimental.pallas import tpu as pltpu

_STRIDES = [8, 16, 32, 64, 128]
_THRESHOLD = 0.99
_HWS = [64, 32, 16, 8, 4]
_NLOC = [hw * hw for hw in _HWS]
_OFFS = [0, 4096, 5120, 5376, 5440]
_TOT = 5456
_C = 80
_OUTC = 87


def _placement(rows, col0):
    r = jax.lax.broadcasted_iota(jnp.int32, (rows, _OUTC), 0)
    c = jax.lax.broadcasted_iota(jnp.int32, (rows, _OUTC), 1)
    return (c == r + col0).astype(jnp.float32)


def _small_placement():
    r = jax.lax.broadcasted_iota(jnp.int32, (8, _OUTC), 0)
    c = jax.lax.broadcasted_iota(jnp.int32, (8, _OUTC), 1)
    e = (c == r + 82) & (r < 5)
    e = e | ((r == 5) & (c == 0)) | ((r == 6) & (c == 1))
    return e.astype(jnp.float32)


def _body(c0, c1, c2, c3, c4, b0, b1, b2, b3, b4, t0, t1, t2, t3, t4,
          out_ref, small_ref):
    cls_refs = [c0, c1, c2, c3, c4]
    bbox_refs = [b0, b1, b2, b3, b4]
    ctr_refs = [t0, t1, t2, t3, t4]
    dn = (((0,), (0,)), ((), ()))
    small_ref[7:8, :] = jnp.zeros((1, _NLOC[0]), jnp.float32)
    for l in range(5):
        m = _NLOC[l]
        hw = _HWS[l]
        x = cls_refs[l][0]
        maxs = jnp.max(x, axis=0, keepdims=True)
        mask = (maxs > _THRESHOLD).astype(jnp.float32)
        im = jax.lax.broadcasted_iota(jnp.int32, (1, m), 1)
        xs = ((im % hw) * _STRIDES[l]).astype(jnp.float32)
        ys = ((im // hw) * _STRIDES[l]).astype(jnp.float32)
        small_ref[0:4, 0:m] = bbox_refs[l][0] * mask
        small_ref[4:5, 0:m] = ctr_refs[l][0] * mask
        small_ref[5:6, 0:m] = xs * mask
        small_ref[6:7, 0:m] = ys * mask
        t = jax.lax.dot_general(x * mask, _placement(_C, 2), dn,
                                preferred_element_type=jnp.float32)
        t += jax.lax.dot_general(small_ref[:, 0:m], _small_placement(), dn,
                                 preferred_element_type=jnp.float32)
        out_ref[0, _OFFS[l]:_OFFS[l] + m, :] = t


def kernel(cls_scores_0, cls_scores_1, cls_scores_2, cls_scores_3, cls_scores_4,
           bbox_preds_0, bbox_preds_1, bbox_preds_2, bbox_preds_3, bbox_preds_4,
           centernesses_0, centernesses_1, centernesses_2, centernesses_3,
           centernesses_4):
    n = cls_scores_0.shape[0]
    cls_l = [cls_scores_0, cls_scores_1, cls_scores_2, cls_scores_3, cls_scores_4]
    bbox_l = [bbox_preds_0, bbox_preds_1, bbox_preds_2, bbox_preds_3, bbox_preds_4]
    ctr_l = [centernesses_0, centernesses_1, centernesses_2, centernesses_3,
             centernesses_4]
    args = []
    specs = []
    for lst, ch in ((cls_l, _C), (bbox_l, 4), (ctr_l, 1)):
        for l in range(5):
            args.append(lst[l].reshape(n, ch, _NLOC[l]))
            specs.append(pl.BlockSpec((1, ch, _NLOC[l]), lambda i: (i, 0, 0)))
    return pl.pallas_call(
        _body,
        grid=(n,),
        in_specs=specs,
        out_specs=pl.BlockSpec((1, _TOT, _OUTC), lambda i: (i, 0, 0)),
        out_shape=jax.ShapeDtypeStruct((n, _TOT, _OUTC), jnp.float32),
        scratch_shapes=[pltpu.VMEM((8, _NLOC[0]), jnp.float32)],
    )(*args)


# speedup vs baseline: 2.8038x; 1.3784x over previous
import jax
import jax.numpy as jnp
from jax.experimental import pallas as pl
from jax.experimental.pallas import tpu as pltpu

_STRIDES = [8, 16, 32, 64, 128]
_THRESHOLD = 0.99
_HWS = [64, 32, 16, 8, 4]
_NLOC = [hw * hw for hw in _HWS]
_OFFS = [0, 4096, 5120, 5376, 5440]
_TOT = 5456
_C = 80
_OUTC = 87


def _cls_placement():
    r = jax.lax.broadcasted_iota(jnp.int32, (_OUTC, _C), 0)
    c = jax.lax.broadcasted_iota(jnp.int32, (_OUTC, _C), 1)
    return (r == c + 2).astype(jnp.float32)


def _small_placement_t():
    r = jax.lax.broadcasted_iota(jnp.int32, (_OUTC, 8), 0)
    c = jax.lax.broadcasted_iota(jnp.int32, (_OUTC, 8), 1)
    e = (r == c + 82) & (c < 5)
    e = e | ((c == 5) & (r == 0)) | ((c == 6) & (r == 1))
    return e.astype(jnp.float32)


def _body(c0, c1, c2, c3, c4, b0, b1, b2, b3, b4, t0, t1, t2, t3, t4,
          out_ref, small_ref):
    cls_refs = [c0, c1, c2, c3, c4]
    bbox_refs = [b0, b1, b2, b3, b4]
    ctr_refs = [t0, t1, t2, t3, t4]
    dt = (((1,), (1,)), ((), ()))
    ds = (((1,), (0,)), ((), ()))
    small_ref[7:8, :] = jnp.zeros((1, _NLOC[0]), jnp.float32)
    for l in range(5):
        m = _NLOC[l]
        hw = _HWS[l]
        v = cls_refs[l][0]
        ind = (v > _THRESHOLD).astype(jnp.float32)
        srow = jax.lax.dot_general(jnp.ones((1, _C), jnp.float32), ind, dt,
                                   preferred_element_type=jnp.float32)
        mask = (srow > 0.0).astype(jnp.float32)
        big = jax.lax.dot_general(_cls_placement(), v, dt,
                                  preferred_element_type=jnp.float32)
        im = jax.lax.broadcasted_iota(jnp.int32, (1, m), 1)
        small_ref[0:4, 0:m] = bbox_refs[l][0]
        small_ref[4:5, 0:m] = ctr_refs[l][0]
        small_ref[5:6, 0:m] = ((im % hw) * _STRIDES[l]).astype(jnp.float32)
        small_ref[6:7, 0:m] = ((im // hw) * _STRIDES[l]).astype(jnp.float32)
        t = jax.lax.dot_general(_small_placement_t(), small_ref[:, 0:m], ds,
                                preferred_element_type=jnp.float32)
        out_ref[0, :, pl.ds(_OFFS[l], m)] = (big + t) * mask


def kernel(cls_scores_0, cls_scores_1, cls_scores_2, cls_scores_3, cls_scores_4,
           bbox_preds_0, bbox_preds_1, bbox_preds_2, bbox_preds_3, bbox_preds_4,
           centernesses_0, centernesses_1, centernesses_2, centernesses_3,
           centernesses_4):
    n = cls_scores_0.shape[0]
    cls_l = [cls_scores_0, cls_scores_1, cls_scores_2, cls_scores_3, cls_scores_4]
    bbox_l = [bbox_preds_0, bbox_preds_1, bbox_preds_2, bbox_preds_3, bbox_preds_4]
    ctr_l = [centernesses_0, centernesses_1, centernesses_2, centernesses_3,
             centernesses_4]
    args = []
    specs = []
    for l in range(5):
        m = _NLOC[l]
        args.append(jnp.transpose(cls_l[l], (0, 2, 3, 1)).reshape(n, m, _C))
        specs.append(pl.BlockSpec((1, m, _C), lambda i: (i, 0, 0)))
    for lst, ch in ((bbox_l, 4), (ctr_l, 1)):
        for l in range(5):
            args.append(lst[l].reshape(n, ch, _NLOC[l]))
            specs.append(pl.BlockSpec((1, ch, _NLOC[l]), lambda i: (i, 0, 0)))
    out = pl.pallas_call(
        _body,
        grid=(n,),
        in_specs=specs,
        out_specs=pl.BlockSpec((1, _OUTC, _TOT), lambda i: (i, 0, 0)),
        out_shape=jax.ShapeDtypeStruct((n, _OUTC, _TOT), jnp.float32),
        scratch_shapes=[pltpu.VMEM((8, _NLOC[0]), jnp.float32)],
    )(*args)
    return jnp.transpose(out, (0, 2, 1))
